# Initial kernel scaffold; baseline (speedup 1.0000x reference)
import jax
import jax.numpy as jnp
from jax.experimental import pallas as pl


def kernel(x, edge_index, batch, W1, b1, W2, b2, W3, b3, W4, b4, W5, b5, g1, be1, g2, be2, g3, be3, g4, be4):
    raise NotImplementedError("write your pallas kernel here")


# speedup vs baseline: 6.4667x; 6.4667x over previous
import functools

import jax
import jax.numpy as jnp
from jax import lax
from jax.experimental import pallas as pl
from jax.experimental.pallas import tpu as pltpu
from jax.experimental.pallas import tpu_sc as plsc

N = 10000
E = 320000
NSUB = 16
NCORE = 2
CHUNK = 128
DH = 128
NWORK = NSUB * NCORE
E_PAD = ((E + NWORK * CHUNK - 1) // (NWORK * CHUNK)) * (NWORK * CHUNK)
ACC_PER_SUB = 632
N_ACC = ACC_PER_SUB * NSUB
OUT_PER_SUB = 624
N_PAD_ROWS = N_ACC - N
ROWS_MM = 1000


def _segsum_sc(table, srcp, dstp, zrows, feat_split):
  mesh = plsc.VectorSubcoreMesh(core_axis_name="c", subcore_axis_name="s")
  eps = E_PAD // NSUB if feat_split else E_PAD // NWORK
  nchunk = eps // CHUNK

  @functools.partial(
      pl.kernel,
      out_type=jax.ShapeDtypeStruct((NCORE, N, DH), jnp.float32),
      mesh=mesh,
      scratch_types=[
          pltpu.VMEM((CHUNK,), jnp.int32),
          pltpu.VMEM((CHUNK,), jnp.int32),
          pltpu.VMEM((CHUNK, DH), jnp.float32),
          pltpu.VMEM_SHARED((N_ACC, DH), jnp.float32),
          pltpu.SemaphoreType.DMA,
      ],
  )
  def k(tab_hbm, src_hbm, dst_hbm, z_hbm, out_hbm, src_v, dst_v, rows_v, acc,
        sem):
    cid = lax.axis_index("c")
    sid = lax.axis_index("s")
    pltpu.sync_copy(z_hbm, acc.at[pl.ds(sid * ACC_PER_SUB, ACC_PER_SUB)])
    plsc.subcore_barrier()
    if feat_split:
      base = sid * eps
      tab = tab_hbm.at[cid]
    else:
      base = (cid * NSUB + sid) * eps
      tab = tab_hbm

    @pl.loop(0, nchunk)
    def _(i):
      off = base + i * CHUNK
      pltpu.sync_copy(src_hbm.at[pl.ds(off, CHUNK)], src_v)
      pltpu.sync_copy(dst_hbm.at[pl.ds(off, CHUNK)], dst_v)
      pltpu.async_copy(tab.at[src_v], rows_v, sem).wait()
      pltpu.sync_copy(rows_v, acc.at[dst_v], add=True)

    plsc.subcore_barrier()
    pltpu.sync_copy(
        acc.at[pl.ds(sid * OUT_PER_SUB, OUT_PER_SUB)],
        out_hbm.at[cid].at[pl.ds(sid * OUT_PER_SUB, OUT_PER_SUB)])

    @pl.when(sid == NSUB - 1)
    def _():
      tail = NSUB * OUT_PER_SUB
      pltpu.sync_copy(acc.at[pl.ds(tail, N - tail)],
                      out_hbm.at[cid].at[pl.ds(tail, N - tail)])

  return k(table, srcp, dstp, zrows)


def _layer_mm(xs, nb, deg, Ah, Bh, bias, dout, stats, feat_split):
  nblk = N // ROWS_MM
  dot = functools.partial(jnp.dot, preferred_element_type=jnp.float32,
                          precision=lax.Precision.HIGHEST)

  def body(xs_ref, nb_ref, deg_ref, a_ref, b2_ref, bias_ref, y_ref, *s_refs):
    deg_blk = deg_ref[...]
    inv = 1.0 / jnp.maximum(deg_blk, 1.0)
    if feat_split:
      y = (dot(xs_ref[0], a_ref[0]) + dot(xs_ref[1], a_ref[1]) +
           dot(nb_ref[0] * inv, b2_ref[0]) + dot(nb_ref[1] * inv, b2_ref[1]))
    else:
      nbm = (nb_ref[0] + nb_ref[1]) * inv
      y = dot(xs_ref[...], a_ref[...]) + dot(nbm, b2_ref[...])
    y = jnp.where(deg_blk > 0.0, y + bias_ref[...], 0.0)
    y_ref[...] = y
    if stats:
      s1_ref, s2_ref = s_refs
      p1 = jnp.sum(y, axis=0, keepdims=True)
      p2 = jnp.sum(y * y, axis=0, keepdims=True)

      @pl.when(pl.program_id(0) == 0)
      def _():
        s1_ref[...] = p1
        s2_ref[...] = p2

      @pl.when(pl.program_id(0) != 0)
      def _():
        s1_ref[...] += p1
        s2_ref[...] += p2

  out_shape = [jax.ShapeDtypeStruct((N, dout), jnp.float32)]
  out_specs = [pl.BlockSpec((ROWS_MM, dout), lambda i: (i, 0))]
  if stats:
    out_shape += [jax.ShapeDtypeStruct((1, dout), jnp.float32)] * 2
    out_specs += [pl.BlockSpec((1, dout), lambda i: (0, 0))] * 2
  if feat_split:
    x_spec = pl.BlockSpec((2, ROWS_MM, DH), lambda i: (0, i, 0))
    w_spec = pl.BlockSpec((2, DH, dout), lambda i: (0, 0, 0))
  else:
    x_spec = pl.BlockSpec((ROWS_MM, DH), lambda i: (i, 0))
    w_spec = pl.BlockSpec((DH, dout), lambda i: (0, 0))
  in_specs = [
      x_spec,
      pl.BlockSpec((2, ROWS_MM, DH), lambda i: (0, i, 0)),
      pl.BlockSpec((ROWS_MM, 1), lambda i: (i, 0)),
      w_spec,
      w_spec,
      pl.BlockSpec((1, dout), lambda i: (0, 0)),
  ]
  return pl.pallas_call(
      body, grid=(nblk,), in_specs=in_specs, out_specs=out_specs,
      out_shape=out_shape)(xs, nb, deg, Ah, Bh, bias)


def _bn_relu(y, s1, s2, g, be, dout, split):
  nblk = N // ROWS_MM

  def body(y_ref, s1_ref, s2_ref, g_ref, be_ref, o_ref):
    mu = s1_ref[...] / N
    var = s2_ref[...] / N - mu * mu
    scale = g_ref[...] * lax.rsqrt(var + 1e-5)
    z = (y_ref[...] - mu) * scale + be_ref[...]
    z = jnp.maximum(z, 0.0)
    if split:
      o_ref[0] = z[:, :DH]
      o_ref[1] = z[:, DH:]
    elif dout == DH:
      o_ref[...] = z
    else:
      o_ref[...] = jnp.concatenate(
          [z, jnp.zeros((ROWS_MM, DH - dout), jnp.float32)], axis=-1)

  in_specs = [
      pl.BlockSpec((ROWS_MM, dout), lambda i: (i, 0)),
      pl.BlockSpec((1, dout), lambda i: (0, 0)),
      pl.BlockSpec((1, dout), lambda i: (0, 0)),
      pl.BlockSpec((1, dout), lambda i: (0, 0)),
      pl.BlockSpec((1, dout), lambda i: (0, 0)),
  ]
  if split:
    out_spec = pl.BlockSpec((2, ROWS_MM, DH), lambda i: (0, i, 0))
    out_shape = jax.ShapeDtypeStruct((NCORE, N, DH), jnp.float32)
  else:
    out_spec = pl.BlockSpec((ROWS_MM, DH), lambda i: (i, 0))
    out_shape = jax.ShapeDtypeStruct((N, DH), jnp.float32)
  return pl.pallas_call(
      body, grid=(nblk,), in_specs=in_specs, out_specs=out_spec,
      out_shape=out_shape)(y, s1, s2, g, be)


def _embed_split(x):
  def body(x_ref, o_ref):
    v = x_ref[...]
    parts = []
    for s in range(3):
      p = v[:, 3 * s:3 * s + 3]
      outs = [p]
      for i in range(10):
        f = 2.0**i
        outs.append(jnp.sin(p * f))
        outs.append(jnp.cos(p * f))
      parts.append(jnp.concatenate(outs, axis=-1))
    rest = v[:, 9:20]
    ones = jnp.ones((ROWS_MM, 1), jnp.float32)
    zer = jnp.zeros((ROWS_MM, 55), jnp.float32)
    h = jnp.concatenate(parts + [rest, ones, zer], axis=-1)
    o_ref[0] = h[:, :DH]
    o_ref[1] = h[:, DH:]

  nblk = N // ROWS_MM
  return pl.pallas_call(
      body, grid=(nblk,),
      in_specs=[pl.BlockSpec((ROWS_MM, 20), lambda i: (i, 0))],
      out_specs=pl.BlockSpec((2, ROWS_MM, DH), lambda i: (0, i, 0)),
      out_shape=jax.ShapeDtypeStruct((NCORE, N, DH), jnp.float32),
  )(x)


def _split_weights(W, d_real, d_pad, feat_split):
  dout = W.shape[0]
  Wa = W[:, :d_real].T
  Wb = W[:, d_real:].T
  A = Wa - Wb
  B = Wb
  if d_pad > d_real:
    A = jnp.pad(A, ((0, d_pad - d_real), (0, 0)))
    B = jnp.pad(B, ((0, d_pad - d_real), (0, 0)))
  if feat_split:
    return A.reshape(2, DH, dout), B.reshape(2, DH, dout)
  return A, B


def kernel(x, edge_index, batch, W1, b1, W2, b2, W3, b3, W4, b4, W5, b5,
           g1, be1, g2, be2, g3, be3, g4, be4):
  del batch
  f32 = jnp.float32
  src = edge_index[0]
  dst = edge_index[1]
  pad = E_PAD - E
  iot = jnp.arange(pad, dtype=jnp.int32)
  srcp = jnp.concatenate([src, iot % CHUNK])
  dstp = jnp.concatenate([dst, N + iot % N_PAD_ROWS])
  zrows = jnp.zeros((ACC_PER_SUB, DH), f32)

  A1, B1 = _split_weights(W1, 200, 256, True)
  A2, B2 = _split_weights(W2, 64, 128, False)
  A3, B3 = _split_weights(W3, 128, 128, False)
  A4, B4 = _split_weights(W4, 256, 256, True)
  A5, B5 = _split_weights(W5, 256, 256, True)
  row = lambda v: v.reshape(1, -1)

  h0 = _embed_split(x)
  nb1 = _segsum_sc(h0, srcp, dstp, zrows, True)
  deg = nb1[1, :, 72:73]
  y1, s1, q1 = _layer_mm(h0, nb1, deg, A1, B1, row(b1), 64, True, True)
  h1 = _bn_relu(y1, s1, q1, row(g1), row(be1), 64, False)
  nb2 = _segsum_sc(h1, srcp, dstp, zrows, False)
  y2, s2, q2 = _layer_mm(h1, nb2, deg, A2, B2, row(b2), 128, True, False)
  h2 = _bn_relu(y2, s2, q2, row(g2), row(be2), 128, False)
  nb3 = _segsum_sc(h2, srcp, dstp, zrows, False)
  y3, s3, q3 = _layer_mm(h2, nb3, deg, A3, B3, row(b3), 256, True, False)
  h3 = _bn_relu(y3, s3, q3, row(g3), row(be3), 256, True)
  nb4 = _segsum_sc(h3, srcp, dstp, zrows, True)
  y4, s4, q4 = _layer_mm(h3, nb4, deg, A4, B4, row(b4), 256, True, True)
  h4 = _bn_relu(y4, s4, q4, row(g4), row(be4), 256, True)
  nb5 = _segsum_sc(h4, srcp, dstp, zrows, True)
  (y5,) = _layer_mm(h4, nb5, deg, A5, B5, row(b5), 512, False, True)
  return y5


# speedup vs baseline: 10.3026x; 1.5932x over previous
import functools

import jax
import jax.numpy as jnp
from jax import lax
from jax.experimental import pallas as pl
from jax.experimental.pallas import tpu as pltpu
from jax.experimental.pallas import tpu_sc as plsc

N = 10000
E = 320000
NSUB = 16
NCORE = 2
CHUNK = 128
DH = 128
NWORK = NSUB * NCORE
NBUF = 4
_ALIGN = NWORK * CHUNK * NBUF
E_PAD = ((E + _ALIGN - 1) // _ALIGN) * _ALIGN
TOT_CHUNK = E_PAD // CHUNK
ACC_PER_SUB = 632
N_ACC = ACC_PER_SUB * NSUB
OUT_PER_SUB = 624
N_PAD_ROWS = N_ACC - N
ROWS_MM = 1000


def _segsum_sc(table, srcp, dstp, zrows, feat_split):
  mesh = plsc.VectorSubcoreMesh(core_axis_name="c", subcore_axis_name="s")
  nchunk = TOT_CHUNK // NSUB if feat_split else TOT_CHUNK // NWORK

  @functools.partial(
      pl.kernel,
      out_type=jax.ShapeDtypeStruct((NCORE, N, DH), jnp.float32),
      mesh=mesh,
      scratch_types=[
          pltpu.VMEM((NBUF, CHUNK), jnp.int32),
          pltpu.VMEM((NBUF, CHUNK), jnp.int32),
          pltpu.VMEM((CHUNK, DH), jnp.float32),
          pltpu.VMEM((CHUNK, DH), jnp.float32),
          pltpu.VMEM_SHARED((N_ACC, DH), jnp.float32),
      ] + [pltpu.SemaphoreType.DMA] * (NBUF + 4),
  )
  def k(tab_hbm, src_hbm, dst_hbm, z_hbm, out_hbm, src_sl, dst_sl,
        rows0, rows1, acc, i0, i1, i2, i3, g_0, g_1, s_0, s_1):
    rows = (rows0, rows1)
    isem = (i0, i1, i2, i3)
    gsem = (g_0, g_1)
    ssem = (s_0, s_1)
    cid = lax.axis_index("c")
    sid = lax.axis_index("s")
    if feat_split:
      cbase = sid * nchunk
      tab = tab_hbm.at[cid]
    else:
      cbase = (cid * NSUB + sid) * nchunk
      tab = tab_hbm

    def fire_idx(c, slot):
      pltpu.async_copy(src_hbm.at[cbase + c], src_sl.at[slot], isem[slot])
      pltpu.async_copy(dst_hbm.at[cbase + c], dst_sl.at[slot], isem[slot])

    def drain_idx(slot):
      pltpu.make_async_copy(src_hbm.at[0], src_sl.at[0], isem[slot]).wait()
      pltpu.make_async_copy(src_hbm.at[0], src_sl.at[0], isem[slot]).wait()

    def drain_rows(sem):
      pltpu.make_async_copy(tab.at[pl.ds(0, CHUNK)], rows[0], sem).wait()

    fire_idx(0, 0)
    fire_idx(1, 1)
    pltpu.sync_copy(z_hbm, acc.at[pl.ds(sid * ACC_PER_SUB, ACC_PER_SUB)])
    plsc.subcore_barrier()
    drain_idx(0)
    pltpu.async_copy(tab.at[src_sl.at[0]], rows[0], gsem[0])

    @pl.loop(0, nchunk, step=NBUF)
    def _(g):
      for u in range(NBUF):
        c = g + u
        rb = u % 2
        rb1 = (u + 1) % 2
        ib1 = (u + 1) % NBUF
        ib2 = (u + 2) % NBUF
        drain_rows(gsem[rb])
        pltpu.async_copy(rows[rb], acc.at[dst_sl.at[u]], ssem[rb], add=True)

        @pl.when(c >= 1)
        def _():
          drain_rows(ssem[rb1])

        @pl.when(c + 2 < nchunk)
        def _():
          fire_idx(c + 2, ib2)

        @pl.when(c + 1 < nchunk)
        def _():
          drain_idx(ib1)
          pltpu.async_copy(tab.at[src_sl.at[ib1]], rows[rb1], gsem[rb1])

    drain_rows(ssem[(nchunk - 1) % 2])
    plsc.subcore_barrier()
    pltpu.sync_copy(
        acc.at[pl.ds(sid * OUT_PER_SUB, OUT_PER_SUB)],
        out_hbm.at[cid].at[pl.ds(sid * OUT_PER_SUB, OUT_PER_SUB)])

    @pl.when(sid == NSUB - 1)
    def _():
      tail = NSUB * OUT_PER_SUB
      pltpu.sync_copy(acc.at[pl.ds(tail, N - tail)],
                      out_hbm.at[cid].at[pl.ds(tail, N - tail)])

  return k(table, srcp, dstp, zrows)


def _layer_mm(xs, nb, deg, Ah, Bh, bias, dout, stats, feat_split):
  nblk = N // ROWS_MM
  dot = functools.partial(jnp.dot, preferred_element_type=jnp.float32,
                          precision=lax.Precision.HIGHEST)

  def body(xs_ref, nb_ref, deg_ref, a_ref, b2_ref, bias_ref, y_ref, *s_refs):
    deg_blk = deg_ref[...]
    inv = 1.0 / jnp.maximum(deg_blk, 1.0)
    if feat_split:
      y = (dot(xs_ref[0], a_ref[0]) + dot(xs_ref[1], a_ref[1]) +
           dot(nb_ref[0] * inv, b2_ref[0]) + dot(nb_ref[1] * inv, b2_ref[1]))
    else:
      nbm = (nb_ref[0] + nb_ref[1]) * inv
      y = dot(xs_ref[...], a_ref[...]) + dot(nbm, b2_ref[...])
    y = jnp.where(deg_blk > 0.0, y + bias_ref[...], 0.0)
    y_ref[...] = y
    if stats:
      s1_ref, s2_ref = s_refs
      p1 = jnp.sum(y, axis=0, keepdims=True)
      p2 = jnp.sum(y * y, axis=0, keepdims=True)

      @pl.when(pl.program_id(0) == 0)
      def _():
        s1_ref[...] = p1
        s2_ref[...] = p2

      @pl.when(pl.program_id(0) != 0)
      def _():
        s1_ref[...] += p1
        s2_ref[...] += p2

  out_shape = [jax.ShapeDtypeStruct((N, dout), jnp.float32)]
  out_specs = [pl.BlockSpec((ROWS_MM, dout), lambda i: (i, 0))]
  if stats:
    out_shape += [jax.ShapeDtypeStruct((1, dout), jnp.float32)] * 2
    out_specs += [pl.BlockSpec((1, dout), lambda i: (0, 0))] * 2
  if feat_split:
    x_spec = pl.BlockSpec((2, ROWS_MM, DH), lambda i: (0, i, 0))
    w_spec = pl.BlockSpec((2, DH, dout), lambda i: (0, 0, 0))
  else:
    x_spec = pl.BlockSpec((ROWS_MM, DH), lambda i: (i, 0))
    w_spec = pl.BlockSpec((DH, dout), lambda i: (0, 0))
  in_specs = [
      x_spec,
      pl.BlockSpec((2, ROWS_MM, DH), lambda i: (0, i, 0)),
      pl.BlockSpec((ROWS_MM, 1), lambda i: (i, 0)),
      w_spec,
      w_spec,
      pl.BlockSpec((1, dout), lambda i: (0, 0)),
  ]
  return pl.pallas_call(
      body, grid=(nblk,), in_specs=in_specs, out_specs=out_specs,
      out_shape=out_shape)(xs, nb, deg, Ah, Bh, bias)


def _bn_relu(y, s1, s2, g, be, dout, split):
  nblk = N // ROWS_MM

  def body(y_ref, s1_ref, s2_ref, g_ref, be_ref, o_ref):
    mu = s1_ref[...] / N
    var = s2_ref[...] / N - mu * mu
    scale = g_ref[...] * lax.rsqrt(var + 1e-5)
    z = (y_ref[...] - mu) * scale + be_ref[...]
    z = jnp.maximum(z, 0.0)
    if split:
      o_ref[0] = z[:, :DH]
      o_ref[1] = z[:, DH:]
    elif dout == DH:
      o_ref[...] = z
    else:
      o_ref[...] = jnp.concatenate(
          [z, jnp.zeros((ROWS_MM, DH - dout), jnp.float32)], axis=-1)

  in_specs = [
      pl.BlockSpec((ROWS_MM, dout), lambda i: (i, 0)),
      pl.BlockSpec((1, dout), lambda i: (0, 0)),
      pl.BlockSpec((1, dout), lambda i: (0, 0)),
      pl.BlockSpec((1, dout), lambda i: (0, 0)),
      pl.BlockSpec((1, dout), lambda i: (0, 0)),
  ]
  if split:
    out_spec = pl.BlockSpec((2, ROWS_MM, DH), lambda i: (0, i, 0))
    out_shape = jax.ShapeDtypeStruct((NCORE, N, DH), jnp.float32)
  else:
    out_spec = pl.BlockSpec((ROWS_MM, DH), lambda i: (i, 0))
    out_shape = jax.ShapeDtypeStruct((N, DH), jnp.float32)
  return pl.pallas_call(
      body, grid=(nblk,), in_specs=in_specs, out_specs=out_spec,
      out_shape=out_shape)(y, s1, s2, g, be)


def _embed_split(x):
  def body(x_ref, o_ref):
    v = x_ref[...]
    parts = []
    for s in range(3):
      p = v[:, 3 * s:3 * s + 3]
      outs = [p]
      for i in range(10):
        f = 2.0**i
        outs.append(jnp.sin(p * f))
        outs.append(jnp.cos(p * f))
      parts.append(jnp.concatenate(outs, axis=-1))
    rest = v[:, 9:20]
    ones = jnp.ones((ROWS_MM, 1), jnp.float32)
    zer = jnp.zeros((ROWS_MM, 55), jnp.float32)
    h = jnp.concatenate(parts + [rest, ones, zer], axis=-1)
    o_ref[0] = h[:, :DH]
    o_ref[1] = h[:, DH:]

  nblk = N // ROWS_MM
  return pl.pallas_call(
      body, grid=(nblk,),
      in_specs=[pl.BlockSpec((ROWS_MM, 20), lambda i: (i, 0))],
      out_specs=pl.BlockSpec((2, ROWS_MM, DH), lambda i: (0, i, 0)),
      out_shape=jax.ShapeDtypeStruct((NCORE, N, DH), jnp.float32),
  )(x)


def _split_weights(W, d_real, d_pad, feat_split):
  dout = W.shape[0]
  Wa = W[:, :d_real].T
  Wb = W[:, d_real:].T
  A = Wa - Wb
  B = Wb
  if d_pad > d_real:
    A = jnp.pad(A, ((0, d_pad - d_real), (0, 0)))
    B = jnp.pad(B, ((0, d_pad - d_real), (0, 0)))
  if feat_split:
    return A.reshape(2, DH, dout), B.reshape(2, DH, dout)
  return A, B


def kernel(x, edge_index, batch, W1, b1, W2, b2, W3, b3, W4, b4, W5, b5,
           g1, be1, g2, be2, g3, be3, g4, be4):
  del batch
  f32 = jnp.float32
  src = edge_index[0]
  dst = edge_index[1]
  pad = E_PAD - E
  iot = jnp.arange(pad, dtype=jnp.int32)
  srcp = jnp.concatenate([src, iot % 4096])
  dstp = jnp.concatenate([dst, N + iot % N_PAD_ROWS])
  srcp = srcp.reshape(TOT_CHUNK, CHUNK)
  dstp = dstp.reshape(TOT_CHUNK, CHUNK)
  zrows = jnp.zeros((ACC_PER_SUB, DH), f32)

  A1, B1 = _split_weights(W1, 200, 256, True)
  A2, B2 = _split_weights(W2, 64, 128, False)
  A3, B3 = _split_weights(W3, 128, 128, False)
  A4, B4 = _split_weights(W4, 256, 256, True)
  A5, B5 = _split_weights(W5, 256, 256, True)
  row = lambda v: v.reshape(1, -1)

  h0 = _embed_split(x)
  nb1 = _segsum_sc(h0, srcp, dstp, zrows, True)
  deg = nb1[1, :, 72:73]
  y1, s1, q1 = _layer_mm(h0, nb1, deg, A1, B1, row(b1), 64, True, True)
  h1 = _bn_relu(y1, s1, q1, row(g1), row(be1), 64, False)
  nb2 = _segsum_sc(h1, srcp, dstp, zrows, False)
  y2, s2, q2 = _layer_mm(h1, nb2, deg, A2, B2, row(b2), 128, True, False)
  h2 = _bn_relu(y2, s2, q2, row(g2), row(be2), 128, False)
  nb3 = _segsum_sc(h2, srcp, dstp, zrows, False)
  y3, s3, q3 = _layer_mm(h2, nb3, deg, A3, B3, row(b3), 256, True, False)
  h3 = _bn_relu(y3, s3, q3, row(g3), row(be3), 256, True)
  nb4 = _segsum_sc(h3, srcp, dstp, zrows, True)
  y4, s4, q4 = _layer_mm(h3, nb4, deg, A4, B4, row(b4), 256, True, True)
  h4 = _bn_relu(y4, s4, q4, row(g4), row(be4), 256, True)
  nb5 = _segsum_sc(h4, srcp, dstp, zrows, True)
  (y5,) = _layer_mm(h4, nb5, deg, A5, B5, row(b5), 512, False, True)
  return y5


# speedup vs baseline: 12.1571x; 1.1800x over previous
import functools

import numpy as np
import jax
import jax.numpy as jnp
from jax import lax
from jax.experimental import pallas as pl
from jax.experimental.pallas import tpu as pltpu
from jax.experimental.pallas import tpu_sc as plsc

N = 10000
E = 320000
NSUB = 16
NCORE = 2
CHUNK = 128
DH = 128
NWORK = NSUB * NCORE
NBUF = 4
_ALIGN = NWORK * CHUNK * NBUF
E_PAD = ((E + _ALIGN - 1) // _ALIGN) * _ALIGN
TOT_CHUNK = E_PAD // CHUNK
ACC_PER_SUB = 632
N_ACC = ACC_PER_SUB * NSUB
OUT_PER_SUB = 624
N_PAD_ROWS = N_ACC - N
ROWS_MM = 1000


def _segsum_sc(table, srcp, dstp, zrows, feat_split):
  mesh = plsc.VectorSubcoreMesh(core_axis_name="c", subcore_axis_name="s")
  nchunk = TOT_CHUNK // NSUB if feat_split else TOT_CHUNK // NWORK

  @functools.partial(
      pl.kernel,
      out_type=jax.ShapeDtypeStruct((NCORE, N, DH), jnp.float32),
      mesh=mesh,
      scratch_types=[
          pltpu.VMEM((NBUF, CHUNK), jnp.int32),
          pltpu.VMEM((NBUF, CHUNK), jnp.int32),
          pltpu.VMEM((CHUNK, DH), jnp.float32),
          pltpu.VMEM((CHUNK, DH), jnp.float32),
          pltpu.VMEM_SHARED((N_ACC, DH), jnp.float32),
      ] + [pltpu.SemaphoreType.DMA] * (NBUF + 4),
  )
  def k(tab_hbm, src_hbm, dst_hbm, z_hbm, out_hbm, src_sl, dst_sl,
        rows0, rows1, acc, i0, i1, i2, i3, g_0, g_1, s_0, s_1):
    rows = (rows0, rows1)
    isem = (i0, i1, i2, i3)
    gsem = (g_0, g_1)
    ssem = (s_0, s_1)
    cid = lax.axis_index("c")
    sid = lax.axis_index("s")
    if feat_split:
      cbase = sid * nchunk
      tab = tab_hbm.at[cid]
    else:
      cbase = (cid * NSUB + sid) * nchunk
      tab = tab_hbm

    def fire_idx(c, slot):
      pltpu.async_copy(src_hbm.at[cbase + c], src_sl.at[slot], isem[slot])
      pltpu.async_copy(dst_hbm.at[cbase + c], dst_sl.at[slot], isem[slot])

    def drain_idx(slot):
      pltpu.make_async_copy(src_hbm.at[0], src_sl.at[0], isem[slot]).wait()
      pltpu.make_async_copy(src_hbm.at[0], src_sl.at[0], isem[slot]).wait()

    def drain_rows(sem):
      pltpu.make_async_copy(tab.at[pl.ds(0, CHUNK)], rows[0], sem).wait()

    fire_idx(0, 0)
    fire_idx(1, 1)
    pltpu.sync_copy(z_hbm, acc.at[pl.ds(sid * ACC_PER_SUB, ACC_PER_SUB)])
    plsc.subcore_barrier()
    drain_idx(0)
    pltpu.async_copy(tab.at[src_sl.at[0]], rows[0], gsem[0])

    @pl.loop(0, nchunk, step=NBUF)
    def _(g):
      for u in range(NBUF):
        c = g + u
        rb = u % 2
        rb1 = (u + 1) % 2
        ib1 = (u + 1) % NBUF
        ib2 = (u + 2) % NBUF
        drain_rows(gsem[rb])
        pltpu.async_copy(rows[rb], acc.at[dst_sl.at[u]], ssem[rb], add=True)

        @pl.when(c >= 1)
        def _():
          drain_rows(ssem[rb1])

        @pl.when(c + 2 < nchunk)
        def _():
          fire_idx(c + 2, ib2)

        @pl.when(c + 1 < nchunk)
        def _():
          drain_idx(ib1)
          pltpu.async_copy(tab.at[src_sl.at[ib1]], rows[rb1], gsem[rb1])

    drain_rows(ssem[(nchunk - 1) % 2])
    plsc.subcore_barrier()
    pltpu.sync_copy(
        acc.at[pl.ds(sid * OUT_PER_SUB, OUT_PER_SUB)],
        out_hbm.at[cid].at[pl.ds(sid * OUT_PER_SUB, OUT_PER_SUB)])

    @pl.when(sid == NSUB - 1)
    def _():
      tail = NSUB * OUT_PER_SUB
      pltpu.sync_copy(acc.at[pl.ds(tail, N - tail)],
                      out_hbm.at[cid].at[pl.ds(tail, N - tail)])

  return k(table, srcp, dstp, zrows)


def _layer_mm(xs, nb, deg, Ah, Bh, bias, dout, stats, feat_split):
  nblk = N // ROWS_MM
  dot = functools.partial(jnp.dot, preferred_element_type=jnp.float32,
                          precision=lax.Precision.HIGHEST)

  def body(xs_ref, nb_ref, deg_ref, a_ref, b2_ref, bias_ref, y_ref, *s_refs):
    deg_blk = deg_ref[...]
    inv = 1.0 / jnp.maximum(deg_blk, 1.0)
    if feat_split:
      y = (dot(xs_ref[0], a_ref[0]) + dot(xs_ref[1], a_ref[1]) +
           dot(nb_ref[0] * inv, b2_ref[0]) + dot(nb_ref[1] * inv, b2_ref[1]))
    else:
      nbm = (nb_ref[0] + nb_ref[1]) * inv
      y = dot(xs_ref[...], a_ref[...]) + dot(nbm, b2_ref[...])
    y = jnp.where(deg_blk > 0.0, y + bias_ref[...], 0.0)
    y_ref[...] = y
    if stats:
      s1_ref, s2_ref = s_refs
      p1 = jnp.sum(y, axis=0, keepdims=True)
      p2 = jnp.sum(y * y, axis=0, keepdims=True)

      @pl.when(pl.program_id(0) == 0)
      def _():
        s1_ref[...] = p1
        s2_ref[...] = p2

      @pl.when(pl.program_id(0) != 0)
      def _():
        s1_ref[...] += p1
        s2_ref[...] += p2

  out_shape = [jax.ShapeDtypeStruct((N, dout), jnp.float32)]
  out_specs = [pl.BlockSpec((ROWS_MM, dout), lambda i: (i, 0))]
  if stats:
    out_shape += [jax.ShapeDtypeStruct((1, dout), jnp.float32)] * 2
    out_specs += [pl.BlockSpec((1, dout), lambda i: (0, 0))] * 2
  if feat_split:
    x_spec = pl.BlockSpec((2, ROWS_MM, DH), lambda i: (0, i, 0))
    w_spec = pl.BlockSpec((2, DH, dout), lambda i: (0, 0, 0))
  else:
    x_spec = pl.BlockSpec((ROWS_MM, DH), lambda i: (i, 0))
    w_spec = pl.BlockSpec((DH, dout), lambda i: (0, 0))
  in_specs = [
      x_spec,
      pl.BlockSpec((2, ROWS_MM, DH), lambda i: (0, i, 0)),
      pl.BlockSpec((ROWS_MM, 1), lambda i: (i, 0)),
      w_spec,
      w_spec,
      pl.BlockSpec((1, dout), lambda i: (0, 0)),
  ]
  return pl.pallas_call(
      body, grid=(nblk,), in_specs=in_specs, out_specs=out_specs,
      out_shape=out_shape)(xs, nb, deg, Ah, Bh, bias)


def _bn_relu(y, s1, s2, g, be, dout, split):
  nblk = N // ROWS_MM

  def body(y_ref, s1_ref, s2_ref, g_ref, be_ref, o_ref):
    mu = s1_ref[...] / N
    var = s2_ref[...] / N - mu * mu
    scale = g_ref[...] * lax.rsqrt(var + 1e-5)
    z = (y_ref[...] - mu) * scale + be_ref[...]
    z = jnp.maximum(z, 0.0)
    if split:
      o_ref[0] = z[:, :DH]
      o_ref[1] = z[:, DH:]
    elif dout == DH:
      o_ref[...] = z
    else:
      o_ref[...] = jnp.concatenate(
          [z, jnp.zeros((ROWS_MM, DH - dout), jnp.float32)], axis=-1)

  in_specs = [
      pl.BlockSpec((ROWS_MM, dout), lambda i: (i, 0)),
      pl.BlockSpec((1, dout), lambda i: (0, 0)),
      pl.BlockSpec((1, dout), lambda i: (0, 0)),
      pl.BlockSpec((1, dout), lambda i: (0, 0)),
      pl.BlockSpec((1, dout), lambda i: (0, 0)),
  ]
  if split:
    out_spec = pl.BlockSpec((2, ROWS_MM, DH), lambda i: (0, i, 0))
    out_shape = jax.ShapeDtypeStruct((NCORE, N, DH), jnp.float32)
  else:
    out_spec = pl.BlockSpec((ROWS_MM, DH), lambda i: (i, 0))
    out_shape = jax.ShapeDtypeStruct((N, DH), jnp.float32)
  return pl.pallas_call(
      body, grid=(nblk,), in_specs=in_specs, out_specs=out_spec,
      out_shape=out_shape)(y, s1, s2, g, be)


def _embed_consts():
  S = np.zeros((20, 256), np.float32)
  mA = np.zeros((1, 256), np.float32)
  mB = np.zeros((1, 256), np.float32)
  mC = np.zeros((1, 256), np.float32)
  mD = np.zeros((1, 256), np.float32)
  for g in range(3):
    base = 63 * g
    for j in range(3):
      S[3 * g + j, base + j] = 1.0
      mC[0, base + j] = 1.0
      for i in range(10):
        S[3 * g + j, base + 3 + 6 * i + j] = 2.0**i
        mA[0, base + 3 + 6 * i + j] = 1.0
        S[3 * g + j, base + 6 + 6 * i + j] = 2.0**i
        mB[0, base + 6 + 6 * i + j] = 1.0
  for t in range(11):
    S[9 + t, 189 + t] = 1.0
    mC[0, 189 + t] = 1.0
  mD[0, 200] = 1.0
  return S, mA, mB, mC, mD


_EMB = _embed_consts()


def _embed_split(x):
  def body(x_ref, s_ref, a_ref, b_ref, c_ref, d_ref, o_ref):
    v = x_ref[...]
    pre = jnp.dot(v, s_ref[...], preferred_element_type=jnp.float32,
                  precision=lax.Precision.HIGHEST)
    h = (jnp.sin(pre) * a_ref[...] + jnp.cos(pre) * b_ref[...] +
         pre * c_ref[...] + d_ref[...])
    o_ref[0] = h[:, :DH]
    o_ref[1] = h[:, DH:]

  nblk = N // ROWS_MM
  full = lambda shape: pl.BlockSpec(shape, lambda i: tuple(0 for _ in shape))
  return pl.pallas_call(
      body, grid=(nblk,),
      in_specs=[pl.BlockSpec((ROWS_MM, 20), lambda i: (i, 0)),
                full((20, 256)), full((1, 256)), full((1, 256)),
                full((1, 256)), full((1, 256))],
      out_specs=pl.BlockSpec((2, ROWS_MM, DH), lambda i: (0, i, 0)),
      out_shape=jax.ShapeDtypeStruct((NCORE, N, DH), jnp.float32),
  )(x, *(jnp.asarray(m) for m in _EMB))


def _split_weights(W, d_real, d_pad, feat_split):
  dout = W.shape[0]
  Wa = W[:, :d_real].T
  Wb = W[:, d_real:].T
  A = Wa - Wb
  B = Wb
  if d_pad > d_real:
    A = jnp.pad(A, ((0, d_pad - d_real), (0, 0)))
    B = jnp.pad(B, ((0, d_pad - d_real), (0, 0)))
  if feat_split:
    return A.reshape(2, DH, dout), B.reshape(2, DH, dout)
  return A, B


def kernel(x, edge_index, batch, W1, b1, W2, b2, W3, b3, W4, b4, W5, b5,
           g1, be1, g2, be2, g3, be3, g4, be4):
  del batch
  f32 = jnp.float32
  src = edge_index[0]
  dst = edge_index[1]
  pad = E_PAD - E
  iot = jnp.arange(pad, dtype=jnp.int32)
  srcp = jnp.concatenate([src, iot % 4096])
  dstp = jnp.concatenate([dst, N + iot % N_PAD_ROWS])
  srcp = srcp.reshape(TOT_CHUNK, CHUNK)
  dstp = dstp.reshape(TOT_CHUNK, CHUNK)
  zrows = jnp.zeros((ACC_PER_SUB, DH), f32)

  A1, B1 = _split_weights(W1, 200, 256, True)
  A2, B2 = _split_weights(W2, 64, 128, False)
  A3, B3 = _split_weights(W3, 128, 128, False)
  A4, B4 = _split_weights(W4, 256, 256, True)
  A5, B5 = _split_weights(W5, 256, 256, True)
  row = lambda v: v.reshape(1, -1)

  h0 = _embed_split(x)
  nb1 = _segsum_sc(h0, srcp, dstp, zrows, True)
  deg = nb1[1, :, 72:73]
  y1, s1, q1 = _layer_mm(h0, nb1, deg, A1, B1, row(b1), 64, True, True)
  h1 = _bn_relu(y1, s1, q1, row(g1), row(be1), 64, False)
  nb2 = _segsum_sc(h1, srcp, dstp, zrows, False)
  y2, s2, q2 = _layer_mm(h1, nb2, deg, A2, B2, row(b2), 128, True, False)
  h2 = _bn_relu(y2, s2, q2, row(g2), row(be2), 128, False)
  nb3 = _segsum_sc(h2, srcp, dstp, zrows, False)
  y3, s3, q3 = _layer_mm(h2, nb3, deg, A3, B3, row(b3), 256, True, False)
  h3 = _bn_relu(y3, s3, q3, row(g3), row(be3), 256, True)
  nb4 = _segsum_sc(h3, srcp, dstp, zrows, True)
  y4, s4, q4 = _layer_mm(h3, nb4, deg, A4, B4, row(b4), 256, True, True)
  h4 = _bn_relu(y4, s4, q4, row(g4), row(be4), 256, True)
  nb5 = _segsum_sc(h4, srcp, dstp, zrows, True)
  (y5,) = _layer_mm(h4, nb5, deg, A5, B5, row(b5), 512, False, True)
  return y5


# speedup vs baseline: 14.1024x; 1.1600x over previous
import functools

import numpy as np
import jax
import jax.numpy as jnp
from jax import lax
from jax.experimental import pallas as pl
from jax.experimental.pallas import tpu as pltpu
from jax.experimental.pallas import tpu_sc as plsc

N = 10000
E = 320000
NSUB = 16
NCORE = 2
CHUNK = 128
DH = 128
NWORK = NSUB * NCORE
NBUF = 3
_ALIGN = NWORK * CHUNK * NBUF
E_PAD = ((E + _ALIGN - 1) // _ALIGN) * _ALIGN
TOT_CHUNK = E_PAD // CHUNK
N_ACC = N + 16
NZSUB = 4
Z_PER_SUB = N_ACC // NZSUB
OUT_PER_SUB = 624
N_PAD_ROWS = N_ACC - N
ROWS_MM = 1000


def _segsum_sc(table, srcp, dstp, zrows, feat_split):
  mesh = plsc.VectorSubcoreMesh(core_axis_name="c", subcore_axis_name="s")
  nchunk = TOT_CHUNK // NSUB if feat_split else TOT_CHUNK // NWORK

  @functools.partial(
      pl.kernel,
      out_type=jax.ShapeDtypeStruct((NCORE, N, DH), jnp.float32),
      mesh=mesh,
      scratch_types=[
          pltpu.VMEM((NBUF, CHUNK), jnp.int32),
          pltpu.VMEM((NBUF, CHUNK), jnp.int32),
          pltpu.VMEM((CHUNK, DH), jnp.float32),
          pltpu.VMEM((CHUNK, DH), jnp.float32),
          pltpu.VMEM((CHUNK, DH), jnp.float32),
          pltpu.VMEM_SHARED((N_ACC, DH), jnp.float32),
      ] + [pltpu.SemaphoreType.DMA] * (3 * NBUF),
  )
  def k(tab_hbm, src_hbm, dst_hbm, z_hbm, out_hbm, src_sl, dst_sl,
        rows0, rows1, rows2, acc, i0, i1, i2, g_0, g_1, g_2, s_0, s_1, s_2):
    rows = (rows0, rows1, rows2)
    isem = (i0, i1, i2)
    gsem = (g_0, g_1, g_2)
    ssem = (s_0, s_1, s_2)
    cid = lax.axis_index("c")
    sid = lax.axis_index("s")
    if feat_split:
      cbase = sid * nchunk
      tab = tab_hbm.at[cid]
    else:
      cbase = (cid * NSUB + sid) * nchunk
      tab = tab_hbm

    def fire_idx(c, slot):
      pltpu.async_copy(src_hbm.at[cbase + c], src_sl.at[slot], isem[slot])
      pltpu.async_copy(dst_hbm.at[cbase + c], dst_sl.at[slot], isem[slot])

    def drain_idx(slot):
      pltpu.make_async_copy(src_hbm.at[0], src_sl.at[0], isem[slot]).wait()
      pltpu.make_async_copy(src_hbm.at[0], src_sl.at[0], isem[slot]).wait()

    def drain_rows(sem):
      pltpu.make_async_copy(tab.at[pl.ds(0, CHUNK)], rows[0], sem).wait()

    fire_idx(0, 0)
    fire_idx(1, 1)

    @pl.when(sid < NZSUB)
    def _():
      pltpu.sync_copy(z_hbm, acc.at[pl.ds(sid * Z_PER_SUB, Z_PER_SUB)])

    plsc.subcore_barrier()
    drain_idx(0)
    pltpu.async_copy(tab.at[src_sl.at[0]], rows[0], gsem[0])

    @pl.loop(0, nchunk, step=NBUF)
    def _(g):
      for u in range(NBUF):
        c = g + u
        u1 = (u + 1) % NBUF
        u2 = (u + 2) % NBUF

        @pl.when(c + 1 < nchunk)
        def _():
          drain_idx(u1)
          pltpu.async_copy(tab.at[src_sl.at[u1]], rows[u1], gsem[u1])

        drain_rows(gsem[u])
        pltpu.async_copy(rows[u], acc.at[dst_sl.at[u]], ssem[u], add=True)

        @pl.when(c >= 1)
        def _():
          drain_rows(ssem[u2])

        @pl.when(c + 2 < nchunk)
        def _():
          fire_idx(c + 2, u2)

    drain_rows(ssem[(nchunk - 1) % NBUF])
    plsc.subcore_barrier()
    pltpu.sync_copy(
        acc.at[pl.ds(sid * OUT_PER_SUB, OUT_PER_SUB)],
        out_hbm.at[cid].at[pl.ds(sid * OUT_PER_SUB, OUT_PER_SUB)])

    @pl.when(sid == NSUB - 1)
    def _():
      tail = NSUB * OUT_PER_SUB
      pltpu.sync_copy(acc.at[pl.ds(tail, N - tail)],
                      out_hbm.at[cid].at[pl.ds(tail, N - tail)])

  return k(table, srcp, dstp, zrows)


def _layer_mm(xs, nb, deg, Ah, Bh, bias, dout, stats, feat_split):
  nblk = N // ROWS_MM
  dot = functools.partial(jnp.dot, preferred_element_type=jnp.float32,
                          precision=lax.Precision.HIGHEST)

  def body(xs_ref, nb_ref, deg_ref, a_ref, b2_ref, bias_ref, y_ref, *s_refs):
    deg_blk = deg_ref[...]
    inv = 1.0 / jnp.maximum(deg_blk, 1.0)
    if feat_split:
      y = (dot(xs_ref[0], a_ref[0]) + dot(xs_ref[1], a_ref[1]) +
           dot(nb_ref[0] * inv, b2_ref[0]) + dot(nb_ref[1] * inv, b2_ref[1]))
    else:
      nbm = (nb_ref[0] + nb_ref[1]) * inv
      y = dot(xs_ref[...], a_ref[...]) + dot(nbm, b2_ref[...])
    y = jnp.where(deg_blk > 0.0, y + bias_ref[...], 0.0)
    y_ref[...] = y
    if stats:
      s1_ref, s2_ref = s_refs
      p1 = jnp.sum(y, axis=0, keepdims=True)
      p2 = jnp.sum(y * y, axis=0, keepdims=True)

      @pl.when(pl.program_id(0) == 0)
      def _():
        s1_ref[...] = p1
        s2_ref[...] = p2

      @pl.when(pl.program_id(0) != 0)
      def _():
        s1_ref[...] += p1
        s2_ref[...] += p2

  out_shape = [jax.ShapeDtypeStruct((N, dout), jnp.float32)]
  out_specs = [pl.BlockSpec((ROWS_MM, dout), lambda i: (i, 0))]
  if stats:
    out_shape += [jax.ShapeDtypeStruct((1, dout), jnp.float32)] * 2
    out_specs += [pl.BlockSpec((1, dout), lambda i: (0, 0))] * 2
  if feat_split:
    x_spec = pl.BlockSpec((2, ROWS_MM, DH), lambda i: (0, i, 0))
    w_spec = pl.BlockSpec((2, DH, dout), lambda i: (0, 0, 0))
  else:
    x_spec = pl.BlockSpec((ROWS_MM, DH), lambda i: (i, 0))
    w_spec = pl.BlockSpec((DH, dout), lambda i: (0, 0))
  in_specs = [
      x_spec,
      pl.BlockSpec((2, ROWS_MM, DH), lambda i: (0, i, 0)),
      pl.BlockSpec((ROWS_MM, 1), lambda i: (i, 0)),
      w_spec,
      w_spec,
      pl.BlockSpec((1, dout), lambda i: (0, 0)),
  ]
  return pl.pallas_call(
      body, grid=(nblk,), in_specs=in_specs, out_specs=out_specs,
      out_shape=out_shape)(xs, nb, deg, Ah, Bh, bias)


def _bn_relu(y, s1, s2, g, be, dout, split):
  nblk = N // ROWS_MM

  def body(y_ref, s1_ref, s2_ref, g_ref, be_ref, o_ref):
    mu = s1_ref[...] / N
    var = s2_ref[...] / N - mu * mu
    scale = g_ref[...] * lax.rsqrt(var + 1e-5)
    z = (y_ref[...] - mu) * scale + be_ref[...]
    z = jnp.maximum(z, 0.0)
    if split:
      o_ref[0] = z[:, :DH]
      o_ref[1] = z[:, DH:]
    elif dout == DH:
      o_ref[...] = z
    else:
      o_ref[...] = jnp.concatenate(
          [z, jnp.zeros((ROWS_MM, DH - dout), jnp.float32)], axis=-1)

  in_specs = [
      pl.BlockSpec((ROWS_MM, dout), lambda i: (i, 0)),
      pl.BlockSpec((1, dout), lambda i: (0, 0)),
      pl.BlockSpec((1, dout), lambda i: (0, 0)),
      pl.BlockSpec((1, dout), lambda i: (0, 0)),
      pl.BlockSpec((1, dout), lambda i: (0, 0)),
  ]
  if split:
    out_spec = pl.BlockSpec((2, ROWS_MM, DH), lambda i: (0, i, 0))
    out_shape = jax.ShapeDtypeStruct((NCORE, N, DH), jnp.float32)
  else:
    out_spec = pl.BlockSpec((ROWS_MM, DH), lambda i: (i, 0))
    out_shape = jax.ShapeDtypeStruct((N, DH), jnp.float32)
  return pl.pallas_call(
      body, grid=(nblk,), in_specs=in_specs, out_specs=out_spec,
      out_shape=out_shape)(y, s1, s2, g, be)


def _embed_consts():
  S = np.zeros((20, 256), np.float32)
  mA = np.zeros((1, 256), np.float32)
  mB = np.zeros((1, 256), np.float32)
  mC = np.zeros((1, 256), np.float32)
  mD = np.zeros((1, 256), np.float32)
  for g in range(3):
    base = 63 * g
    for j in range(3):
      S[3 * g + j, base + j] = 1.0
      mC[0, base + j] = 1.0
      for i in range(10):
        S[3 * g + j, base + 3 + 6 * i + j] = 2.0**i
        mA[0, base + 3 + 6 * i + j] = 1.0
        S[3 * g + j, base + 6 + 6 * i + j] = 2.0**i
        mB[0, base + 6 + 6 * i + j] = 1.0
  for t in range(11):
    S[9 + t, 189 + t] = 1.0
    mC[0, 189 + t] = 1.0
  mD[0, 200] = 1.0
  return S, mA, mB, mC, mD


_EMB = _embed_consts()


def _embed_split(x):
  def body(x_ref, s_ref, a_ref, b_ref, c_ref, d_ref, o_ref):
    v = x_ref[...]
    pre = jnp.dot(v, s_ref[...], preferred_element_type=jnp.float32,
                  precision=lax.Precision.HIGHEST)
    h = (jnp.sin(pre) * a_ref[...] + jnp.cos(pre) * b_ref[...] +
         pre * c_ref[...] + d_ref[...])
    o_ref[0] = h[:, :DH]
    o_ref[1] = h[:, DH:]

  nblk = N // ROWS_MM
  full = lambda shape: pl.BlockSpec(shape, lambda i: tuple(0 for _ in shape))
  return pl.pallas_call(
      body, grid=(nblk,),
      in_specs=[pl.BlockSpec((ROWS_MM, 20), lambda i: (i, 0)),
                full((20, 256)), full((1, 256)), full((1, 256)),
                full((1, 256)), full((1, 256))],
      out_specs=pl.BlockSpec((2, ROWS_MM, DH), lambda i: (0, i, 0)),
      out_shape=jax.ShapeDtypeStruct((NCORE, N, DH), jnp.float32),
  )(x, *(jnp.asarray(m) for m in _EMB))


def _split_weights(W, d_real, d_pad, feat_split):
  dout = W.shape[0]
  Wa = W[:, :d_real].T
  Wb = W[:, d_real:].T
  A = Wa - Wb
  B = Wb
  if d_pad > d_real:
    A = jnp.pad(A, ((0, d_pad - d_real), (0, 0)))
    B = jnp.pad(B, ((0, d_pad - d_real), (0, 0)))
  if feat_split:
    return A.reshape(2, DH, dout), B.reshape(2, DH, dout)
  return A, B


def kernel(x, edge_index, batch, W1, b1, W2, b2, W3, b3, W4, b4, W5, b5,
           g1, be1, g2, be2, g3, be3, g4, be4):
  del batch
  f32 = jnp.float32
  src = edge_index[0]
  dst = edge_index[1]
  pad = E_PAD - E
  iot = jnp.arange(pad, dtype=jnp.int32)
  srcp = jnp.concatenate([src, iot % 4096])
  dstp = jnp.concatenate([dst, N + iot % N_PAD_ROWS])
  srcp = srcp.reshape(TOT_CHUNK, CHUNK)
  dstp = dstp.reshape(TOT_CHUNK, CHUNK)
  zrows = jnp.zeros((Z_PER_SUB, DH), f32)

  A1, B1 = _split_weights(W1, 200, 256, True)
  A2, B2 = _split_weights(W2, 64, 128, False)
  A3, B3 = _split_weights(W3, 128, 128, False)
  A4, B4 = _split_weights(W4, 256, 256, True)
  A5, B5 = _split_weights(W5, 256, 256, True)
  row = lambda v: v.reshape(1, -1)

  h0 = _embed_split(x)
  nb1 = _segsum_sc(h0, srcp, dstp, zrows, True)
  deg = nb1[1, :, 72:73]
  y1, s1, q1 = _layer_mm(h0, nb1, deg, A1, B1, row(b1), 64, True, True)
  h1 = _bn_relu(y1, s1, q1, row(g1), row(be1), 64, False)
  nb2 = _segsum_sc(h1, srcp, dstp, zrows, False)
  y2, s2, q2 = _layer_mm(h1, nb2, deg, A2, B2, row(b2), 128, True, False)
  h2 = _bn_relu(y2, s2, q2, row(g2), row(be2), 128, False)
  nb3 = _segsum_sc(h2, srcp, dstp, zrows, False)
  y3, s3, q3 = _layer_mm(h2, nb3, deg, A3, B3, row(b3), 256, True, False)
  h3 = _bn_relu(y3, s3, q3, row(g3), row(be3), 256, True)
  nb4 = _segsum_sc(h3, srcp, dstp, zrows, True)
  y4, s4, q4 = _layer_mm(h3, nb4, deg, A4, B4, row(b4), 256, True, True)
  h4 = _bn_relu(y4, s4, q4, row(g4), row(be4), 256, True)
  nb5 = _segsum_sc(h4, srcp, dstp, zrows, True)
  (y5,) = _layer_mm(h4, nb5, deg, A5, B5, row(b5), 512, False, True)
  return y5


# speedup vs baseline: 14.8948x; 1.0562x over previous
import functools

import numpy as np
import jax
import jax.numpy as jnp
from jax import lax
from jax.experimental import pallas as pl
from jax.experimental.pallas import tpu as pltpu
from jax.experimental.pallas import tpu_sc as plsc

N = 10000
E = 320000
NSUB = 16
NCORE = 2
CHUNK = 128
DH = 128
NWORK = NSUB * NCORE
NBUF = 3
_ALIGN = NWORK * CHUNK * NBUF
E_PAD = ((E + _ALIGN - 1) // _ALIGN) * _ALIGN
TOT_CHUNK = E_PAD // CHUNK
N_ACC = N + 16
NZSUB = 4
Z_PER_SUB = N_ACC // NZSUB
OUT_PER_SUB = 624
N_PAD_ROWS = N_ACC - N
ROWS_MM = 1000


def _segsum_sc(table, srcp, dstp, zrows, feat_split):
  mesh = plsc.VectorSubcoreMesh(core_axis_name="c", subcore_axis_name="s")
  nchunk = TOT_CHUNK // NSUB if feat_split else TOT_CHUNK // NWORK

  @functools.partial(
      pl.kernel,
      out_type=jax.ShapeDtypeStruct((NCORE, N, DH), jnp.float32),
      mesh=mesh,
      scratch_types=[
          pltpu.VMEM((NBUF, CHUNK), jnp.int32),
          pltpu.VMEM((NBUF, CHUNK), jnp.int32),
          pltpu.VMEM((CHUNK, DH), jnp.float32),
          pltpu.VMEM((CHUNK, DH), jnp.float32),
          pltpu.VMEM((CHUNK, DH), jnp.float32),
          pltpu.VMEM_SHARED((N_ACC, DH), jnp.float32),
      ] + [pltpu.SemaphoreType.DMA] * (3 * NBUF),
  )
  def k(tab_hbm, src_hbm, dst_hbm, z_hbm, out_hbm, src_sl, dst_sl,
        rows0, rows1, rows2, acc, i0, i1, i2, g_0, g_1, g_2, s_0, s_1, s_2):
    rows = (rows0, rows1, rows2)
    isem = (i0, i1, i2)
    gsem = (g_0, g_1, g_2)
    ssem = (s_0, s_1, s_2)
    cid = lax.axis_index("c")
    sid = lax.axis_index("s")
    if feat_split:
      cbase = sid * nchunk
      tab = tab_hbm.at[cid]
    else:
      cbase = (cid * NSUB + sid) * nchunk
      tab = tab_hbm

    def fire_idx(c, slot):
      pltpu.async_copy(src_hbm.at[cbase + c], src_sl.at[slot], isem[slot])
      pltpu.async_copy(dst_hbm.at[cbase + c], dst_sl.at[slot], isem[slot])

    def drain_idx(slot):
      pltpu.make_async_copy(src_hbm.at[0], src_sl.at[0], isem[slot]).wait()
      pltpu.make_async_copy(src_hbm.at[0], src_sl.at[0], isem[slot]).wait()

    def drain_rows(sem):
      pltpu.make_async_copy(tab.at[pl.ds(0, CHUNK)], rows[0], sem).wait()

    fire_idx(0, 0)
    fire_idx(1, 1)

    @pl.when(sid < NZSUB)
    def _():
      pltpu.sync_copy(z_hbm, acc.at[pl.ds(sid * Z_PER_SUB, Z_PER_SUB)])

    plsc.subcore_barrier()
    drain_idx(0)
    pltpu.async_copy(tab.at[src_sl.at[0]], rows[0], gsem[0])

    @pl.loop(0, nchunk, step=NBUF)
    def _(g):
      for u in range(NBUF):
        c = g + u
        u1 = (u + 1) % NBUF
        u2 = (u + 2) % NBUF

        @pl.when(c + 1 < nchunk)
        def _():
          drain_idx(u1)
          pltpu.async_copy(tab.at[src_sl.at[u1]], rows[u1], gsem[u1])

        drain_rows(gsem[u])
        pltpu.async_copy(rows[u], acc.at[dst_sl.at[u]], ssem[u], add=True)

        @pl.when(c >= 1)
        def _():
          drain_rows(ssem[u2])

        @pl.when(c + 2 < nchunk)
        def _():
          fire_idx(c + 2, u2)

    drain_rows(ssem[(nchunk - 1) % NBUF])
    plsc.subcore_barrier()
    pltpu.sync_copy(
        acc.at[pl.ds(sid * OUT_PER_SUB, OUT_PER_SUB)],
        out_hbm.at[cid].at[pl.ds(sid * OUT_PER_SUB, OUT_PER_SUB)])

    @pl.when(sid == NSUB - 1)
    def _():
      tail = NSUB * OUT_PER_SUB
      pltpu.sync_copy(acc.at[pl.ds(tail, N - tail)],
                      out_hbm.at[cid].at[pl.ds(tail, N - tail)])

  return k(table, srcp, dstp, zrows)


_dot = functools.partial(jnp.dot, preferred_element_type=jnp.float32,
                         precision=lax.Precision.HIGHEST)


def _wspec(dout, feat_split):
  if feat_split:
    return pl.BlockSpec((2, DH, dout), lambda i: (0, 0, 0))
  return pl.BlockSpec((DH, dout), lambda i: (0, 0))


def _xspec(feat_split):
  if feat_split:
    return pl.BlockSpec((2, ROWS_MM, DH), lambda i: (0, i, 0))
  return pl.BlockSpec((ROWS_MM, DH), lambda i: (i, 0))


def _pre_mm(xs, Ah, bias, dout, feat_split):
  def body(xs_ref, a_ref, bias_ref, p_ref):
    if feat_split:
      p = _dot(xs_ref[0], a_ref[0]) + _dot(xs_ref[1], a_ref[1])
    else:
      p = _dot(xs_ref[...], a_ref[...])
    p_ref[...] = p + bias_ref[...]

  return pl.pallas_call(
      body, grid=(N // ROWS_MM,),
      in_specs=[_xspec(feat_split), _wspec(dout, feat_split),
                pl.BlockSpec((1, dout), lambda i: (0, 0))],
      out_specs=pl.BlockSpec((ROWS_MM, dout), lambda i: (i, 0)),
      out_shape=jax.ShapeDtypeStruct((N, dout), jnp.float32),
  )(xs, Ah, bias)


def _post_mm(p, nb, deg, Bh, dout, stats, feat_split):
  def body(p_ref, nb_ref, deg_ref, b2_ref, y_ref, *s_refs):
    deg_blk = deg_ref[...]
    inv = 1.0 / jnp.maximum(deg_blk, 1.0)
    if feat_split:
      y = (p_ref[...] + _dot(nb_ref[0] * inv, b2_ref[0]) +
           _dot(nb_ref[1] * inv, b2_ref[1]))
    else:
      y = p_ref[...] + _dot((nb_ref[0] + nb_ref[1]) * inv, b2_ref[...])
    y = jnp.where(deg_blk > 0.0, y, 0.0)
    y_ref[...] = y
    if stats:
      s1_ref, s2_ref = s_refs
      p1 = jnp.sum(y, axis=0, keepdims=True)
      p2 = jnp.sum(y * y, axis=0, keepdims=True)

      @pl.when(pl.program_id(0) == 0)
      def _():
        s1_ref[...] = p1
        s2_ref[...] = p2

      @pl.when(pl.program_id(0) != 0)
      def _():
        s1_ref[...] += p1
        s2_ref[...] += p2

  out_shape = [jax.ShapeDtypeStruct((N, dout), jnp.float32)]
  out_specs = [pl.BlockSpec((ROWS_MM, dout), lambda i: (i, 0))]
  if stats:
    out_shape += [jax.ShapeDtypeStruct((1, dout), jnp.float32)] * 2
    out_specs += [pl.BlockSpec((1, dout), lambda i: (0, 0))] * 2
  in_specs = [
      pl.BlockSpec((ROWS_MM, dout), lambda i: (i, 0)),
      pl.BlockSpec((2, ROWS_MM, DH), lambda i: (0, i, 0)),
      pl.BlockSpec((ROWS_MM, 1), lambda i: (i, 0)),
      _wspec(dout, feat_split),
  ]
  return pl.pallas_call(
      body, grid=(N // ROWS_MM,), in_specs=in_specs, out_specs=out_specs,
      out_shape=out_shape)(p, nb, deg, Bh)


def _bn_relu(y, s1, s2, g, be, dout, split):
  nblk = N // ROWS_MM

  def body(y_ref, s1_ref, s2_ref, g_ref, be_ref, o_ref):
    mu = s1_ref[...] / N
    var = s2_ref[...] / N - mu * mu
    scale = g_ref[...] * lax.rsqrt(var + 1e-5)
    z = (y_ref[...] - mu) * scale + be_ref[...]
    z = jnp.maximum(z, 0.0)
    if split:
      o_ref[0] = z[:, :DH]
      o_ref[1] = z[:, DH:]
    elif dout == DH:
      o_ref[...] = z
    else:
      o_ref[...] = jnp.concatenate(
          [z, jnp.zeros((ROWS_MM, DH - dout), jnp.float32)], axis=-1)

  in_specs = [
      pl.BlockSpec((ROWS_MM, dout), lambda i: (i, 0)),
      pl.BlockSpec((1, dout), lambda i: (0, 0)),
      pl.BlockSpec((1, dout), lambda i: (0, 0)),
      pl.BlockSpec((1, dout), lambda i: (0, 0)),
      pl.BlockSpec((1, dout), lambda i: (0, 0)),
  ]
  if split:
    out_spec = pl.BlockSpec((2, ROWS_MM, DH), lambda i: (0, i, 0))
    out_shape = jax.ShapeDtypeStruct((NCORE, N, DH), jnp.float32)
  else:
    out_spec = pl.BlockSpec((ROWS_MM, DH), lambda i: (i, 0))
    out_shape = jax.ShapeDtypeStruct((N, DH), jnp.float32)
  return pl.pallas_call(
      body, grid=(nblk,), in_specs=in_specs, out_specs=out_spec,
      out_shape=out_shape)(y, s1, s2, g, be)


def _embed_consts():
  S = np.zeros((20, 256), np.float32)
  mA = np.zeros((1, 256), np.float32)
  mB = np.zeros((1, 256), np.float32)
  mC = np.zeros((1, 256), np.float32)
  mD = np.zeros((1, 256), np.float32)
  for g in range(3):
    base = 63 * g
    for j in range(3):
      S[3 * g + j, base + j] = 1.0
      mC[0, base + j] = 1.0
      for i in range(10):
        S[3 * g + j, base + 3 + 6 * i + j] = 2.0**i
        mA[0, base + 3 + 6 * i + j] = 1.0
        S[3 * g + j, base + 6 + 6 * i + j] = 2.0**i
        mB[0, base + 6 + 6 * i + j] = 1.0
  for t in range(11):
    S[9 + t, 189 + t] = 1.0
    mC[0, 189 + t] = 1.0
  mD[0, 200] = 1.0
  return S, mA, mB, mC, mD


_EMB = _embed_consts()


def _embed_split(x):
  def body(x_ref, s_ref, a_ref, b_ref, c_ref, d_ref, o_ref):
    v = x_ref[...]
    pre = jnp.dot(v, s_ref[...], preferred_element_type=jnp.float32,
                  precision=lax.Precision.HIGHEST)
    h = (jnp.sin(pre) * a_ref[...] + jnp.cos(pre) * b_ref[...] +
         pre * c_ref[...] + d_ref[...])
    o_ref[0] = h[:, :DH]
    o_ref[1] = h[:, DH:]

  nblk = N // ROWS_MM
  full = lambda shape: pl.BlockSpec(shape, lambda i: tuple(0 for _ in shape))
  return pl.pallas_call(
      body, grid=(nblk,),
      in_specs=[pl.BlockSpec((ROWS_MM, 20), lambda i: (i, 0)),
                full((20, 256)), full((1, 256)), full((1, 256)),
                full((1, 256)), full((1, 256))],
      out_specs=pl.BlockSpec((2, ROWS_MM, DH), lambda i: (0, i, 0)),
      out_shape=jax.ShapeDtypeStruct((NCORE, N, DH), jnp.float32),
  )(x, *(jnp.asarray(m) for m in _EMB))


def _split_weights(W, d_real, d_pad, feat_split):
  dout = W.shape[0]
  Wa = W[:, :d_real].T
  Wb = W[:, d_real:].T
  A = Wa - Wb
  B = Wb
  if d_pad > d_real:
    A = jnp.pad(A, ((0, d_pad - d_real), (0, 0)))
    B = jnp.pad(B, ((0, d_pad - d_real), (0, 0)))
  if feat_split:
    return A.reshape(2, DH, dout), B.reshape(2, DH, dout)
  return A, B


def kernel(x, edge_index, batch, W1, b1, W2, b2, W3, b3, W4, b4, W5, b5,
           g1, be1, g2, be2, g3, be3, g4, be4):
  del batch
  f32 = jnp.float32
  src = edge_index[0]
  dst = edge_index[1]
  pad = E_PAD - E
  iot = jnp.arange(pad, dtype=jnp.int32)
  srcp = jnp.concatenate([src, iot % 4096])
  dstp = jnp.concatenate([dst, N + iot % N_PAD_ROWS])
  srcp = srcp.reshape(TOT_CHUNK, CHUNK)
  dstp = dstp.reshape(TOT_CHUNK, CHUNK)
  zrows = jnp.zeros((Z_PER_SUB, DH), f32)

  A1, B1 = _split_weights(W1, 200, 256, True)
  A2, B2 = _split_weights(W2, 64, 128, False)
  A3, B3 = _split_weights(W3, 128, 128, False)
  A4, B4 = _split_weights(W4, 256, 256, True)
  A5, B5 = _split_weights(W5, 256, 256, True)
  row = lambda v: v.reshape(1, -1)

  h0 = _embed_split(x)
  nb1 = _segsum_sc(h0, srcp, dstp, zrows, True)
  p1 = _pre_mm(h0, A1, row(b1), 64, True)
  deg = nb1[1, :, 72:73]
  y1, s1, q1 = _post_mm(p1, nb1, deg, B1, 64, True, True)
  h1 = _bn_relu(y1, s1, q1, row(g1), row(be1), 64, False)
  nb2 = _segsum_sc(h1, srcp, dstp, zrows, False)
  p2 = _pre_mm(h1, A2, row(b2), 128, False)
  y2, s2, q2 = _post_mm(p2, nb2, deg, B2, 128, True, False)
  h2 = _bn_relu(y2, s2, q2, row(g2), row(be2), 128, False)
  nb3 = _segsum_sc(h2, srcp, dstp, zrows, False)
  p3 = _pre_mm(h2, A3, row(b3), 256, False)
  y3, s3, q3 = _post_mm(p3, nb3, deg, B3, 256, True, False)
  h3 = _bn_relu(y3, s3, q3, row(g3), row(be3), 256, True)
  nb4 = _segsum_sc(h3, srcp, dstp, zrows, True)
  p4 = _pre_mm(h3, A4, row(b4), 256, True)
  y4, s4, q4 = _post_mm(p4, nb4, deg, B4, 256, True, True)
  h4 = _bn_relu(y4, s4, q4, row(g4), row(be4), 256, True)
  nb5 = _segsum_sc(h4, srcp, dstp, zrows, True)
  p5 = _pre_mm(h4, A5, row(b5), 512, True)
  (y5,) = _post_mm(p5, nb5, deg, B5, 512, False, True)
  return y5


# speedup vs baseline: 15.9124x; 1.0683x over previous
import functools

import numpy as np
import jax
import jax.numpy as jnp
from jax import lax
from jax.experimental import pallas as pl
from jax.experimental.pallas import tpu as pltpu
from jax.experimental.pallas import tpu_sc as plsc

N = 10000
E = 320000
NSUB = 16
NCORE = 2
CHUNK = 96
DH = 128
NWORK = NSUB * NCORE
NBUF = 4
_ALIGN = NWORK * CHUNK * NBUF
E_PAD = ((E + _ALIGN - 1) // _ALIGN) * _ALIGN
TOT_CHUNK = E_PAD // CHUNK
N_ACC = N + 16
NZSUB = 4
Z_PER_SUB = N_ACC // NZSUB
OUT_PER_SUB = 624
N_PAD_ROWS = N_ACC - N
ROWS_MM = 1000


def _segsum_sc(table, srcp, dstp, zrows, feat_split):
  mesh = plsc.VectorSubcoreMesh(core_axis_name="c", subcore_axis_name="s")
  nchunk = TOT_CHUNK // NSUB if feat_split else TOT_CHUNK // NWORK

  @functools.partial(
      pl.kernel,
      out_type=jax.ShapeDtypeStruct((NCORE, N, DH), jnp.float32),
      mesh=mesh,
      scratch_types=[
          pltpu.VMEM((NBUF, CHUNK), jnp.int32),
          pltpu.VMEM((NBUF, CHUNK), jnp.int32),
          pltpu.VMEM((CHUNK, DH), jnp.float32),
          pltpu.VMEM((CHUNK, DH), jnp.float32),
          pltpu.VMEM((CHUNK, DH), jnp.float32),
          pltpu.VMEM((CHUNK, DH), jnp.float32),
          pltpu.VMEM_SHARED((N_ACC, DH), jnp.float32),
      ] + [pltpu.SemaphoreType.DMA] * (3 * NBUF),
  )
  def k(tab_hbm, src_hbm, dst_hbm, z_hbm, out_hbm, src_sl, dst_sl,
        rows0, rows1, rows2, rows3, acc, i0, i1, i2, i3,
        g_0, g_1, g_2, g_3, s_0, s_1, s_2, s_3):
    rows = (rows0, rows1, rows2, rows3)
    isem = (i0, i1, i2, i3)
    gsem = (g_0, g_1, g_2, g_3)
    ssem = (s_0, s_1, s_2, s_3)
    cid = lax.axis_index("c")
    sid = lax.axis_index("s")
    if feat_split:
      cbase = sid * nchunk
      tab = tab_hbm.at[cid]
    else:
      cbase = (cid * NSUB + sid) * nchunk
      tab = tab_hbm

    def fire_idx(c, slot):
      pltpu.async_copy(src_hbm.at[cbase + c], src_sl.at[slot], isem[slot])
      pltpu.async_copy(dst_hbm.at[cbase + c], dst_sl.at[slot], isem[slot])

    def drain_idx(slot):
      pltpu.make_async_copy(src_hbm.at[0], src_sl.at[0], isem[slot]).wait()
      pltpu.make_async_copy(src_hbm.at[0], src_sl.at[0], isem[slot]).wait()

    def drain_rows(sem):
      pltpu.make_async_copy(tab.at[pl.ds(0, CHUNK)], rows[0], sem).wait()

    fire_idx(0, 0)
    fire_idx(1, 1)
    fire_idx(2, 2)

    @pl.when(sid < NZSUB)
    def _():
      pltpu.sync_copy(z_hbm, acc.at[pl.ds(sid * Z_PER_SUB, Z_PER_SUB)])

    plsc.subcore_barrier()
    drain_idx(0)
    pltpu.async_copy(tab.at[src_sl.at[0]], rows[0], gsem[0])
    drain_idx(1)
    pltpu.async_copy(tab.at[src_sl.at[1]], rows[1], gsem[1])

    @pl.loop(0, nchunk, step=NBUF)
    def _(g):
      for u in range(NBUF):
        c = g + u
        u2 = (u + 2) % NBUF
        u3 = (u + 3) % NBUF

        @pl.when(c + 2 < nchunk)
        def _():
          drain_idx(u2)
          pltpu.async_copy(tab.at[src_sl.at[u2]], rows[u2], gsem[u2])

        drain_rows(gsem[u])
        pltpu.async_copy(rows[u], acc.at[dst_sl.at[u]], ssem[u], add=True)

        @pl.when(c >= 1)
        def _():
          drain_rows(ssem[u3])

        @pl.when(c + 3 < nchunk)
        def _():
          fire_idx(c + 3, u3)

    drain_rows(ssem[(nchunk - 1) % NBUF])
    plsc.subcore_barrier()
    pltpu.sync_copy(
        acc.at[pl.ds(sid * OUT_PER_SUB, OUT_PER_SUB)],
        out_hbm.at[cid].at[pl.ds(sid * OUT_PER_SUB, OUT_PER_SUB)])

    @pl.when(sid == NSUB - 1)
    def _():
      tail = NSUB * OUT_PER_SUB
      pltpu.sync_copy(acc.at[pl.ds(tail, N - tail)],
                      out_hbm.at[cid].at[pl.ds(tail, N - tail)])

  return k(table, srcp, dstp, zrows)


_dot = functools.partial(jnp.dot, preferred_element_type=jnp.float32,
                         precision=lax.Precision.HIGHEST)


def _wspec(dout, feat_split):
  if feat_split:
    return pl.BlockSpec((2, DH, dout), lambda i: (0, 0, 0))
  return pl.BlockSpec((DH, dout), lambda i: (0, 0))


def _xspec(feat_split):
  if feat_split:
    return pl.BlockSpec((2, ROWS_MM, DH), lambda i: (0, i, 0))
  return pl.BlockSpec((ROWS_MM, DH), lambda i: (i, 0))


def _pre_mm(xs, Ah, bias, dout, feat_split):
  def body(xs_ref, a_ref, bias_ref, p_ref):
    if feat_split:
      p = _dot(xs_ref[0], a_ref[0]) + _dot(xs_ref[1], a_ref[1])
    else:
      p = _dot(xs_ref[...], a_ref[...])
    p_ref[...] = p + bias_ref[...]

  return pl.pallas_call(
      body, grid=(N // ROWS_MM,),
      in_specs=[_xspec(feat_split), _wspec(dout, feat_split),
                pl.BlockSpec((1, dout), lambda i: (0, 0))],
      out_specs=pl.BlockSpec((ROWS_MM, dout), lambda i: (i, 0)),
      out_shape=jax.ShapeDtypeStruct((N, dout), jnp.float32),
  )(xs, Ah, bias)


def _post_mm(p, nb, deg, Bh, dout, stats, feat_split):
  def body(p_ref, nb_ref, deg_ref, b2_ref, y_ref, *s_refs):
    deg_blk = deg_ref[...]
    inv = 1.0 / jnp.maximum(deg_blk, 1.0)
    if feat_split:
      y = (p_ref[...] + _dot(nb_ref[0] * inv, b2_ref[0]) +
           _dot(nb_ref[1] * inv, b2_ref[1]))
    else:
      y = p_ref[...] + _dot((nb_ref[0] + nb_ref[1]) * inv, b2_ref[...])
    y = jnp.where(deg_blk > 0.0, y, 0.0)
    y_ref[...] = y
    if stats:
      s1_ref, s2_ref = s_refs
      p1 = jnp.sum(y, axis=0, keepdims=True)
      p2 = jnp.sum(y * y, axis=0, keepdims=True)

      @pl.when(pl.program_id(0) == 0)
      def _():
        s1_ref[...] = p1
        s2_ref[...] = p2

      @pl.when(pl.program_id(0) != 0)
      def _():
        s1_ref[...] += p1
        s2_ref[...] += p2

  out_shape = [jax.ShapeDtypeStruct((N, dout), jnp.float32)]
  out_specs = [pl.BlockSpec((ROWS_MM, dout), lambda i: (i, 0))]
  if stats:
    out_shape += [jax.ShapeDtypeStruct((1, dout), jnp.float32)] * 2
    out_specs += [pl.BlockSpec((1, dout), lambda i: (0, 0))] * 2
  in_specs = [
      pl.BlockSpec((ROWS_MM, dout), lambda i: (i, 0)),
      pl.BlockSpec((2, ROWS_MM, DH), lambda i: (0, i, 0)),
      pl.BlockSpec((ROWS_MM, 1), lambda i: (i, 0)),
      _wspec(dout, feat_split),
  ]
  return pl.pallas_call(
      body, grid=(N // ROWS_MM,), in_specs=in_specs, out_specs=out_specs,
      out_shape=out_shape)(p, nb, deg, Bh)


def _bn_relu(y, s1, s2, g, be, dout, split):
  nblk = N // ROWS_MM

  def body(y_ref, s1_ref, s2_ref, g_ref, be_ref, o_ref):
    mu = s1_ref[...] / N
    var = s2_ref[...] / N - mu * mu
    scale = g_ref[...] * lax.rsqrt(var + 1e-5)
    z = (y_ref[...] - mu) * scale + be_ref[...]
    z = jnp.maximum(z, 0.0)
    if split:
      o_ref[0] = z[:, :DH]
      o_ref[1] = z[:, DH:]
    elif dout == DH:
      o_ref[...] = z
    else:
      o_ref[...] = jnp.concatenate(
          [z, jnp.zeros((ROWS_MM, DH - dout), jnp.float32)], axis=-1)

  in_specs = [
      pl.BlockSpec((ROWS_MM, dout), lambda i: (i, 0)),
      pl.BlockSpec((1, dout), lambda i: (0, 0)),
      pl.BlockSpec((1, dout), lambda i: (0, 0)),
      pl.BlockSpec((1, dout), lambda i: (0, 0)),
      pl.BlockSpec((1, dout), lambda i: (0, 0)),
  ]
  if split:
    out_spec = pl.BlockSpec((2, ROWS_MM, DH), lambda i: (0, i, 0))
    out_shape = jax.ShapeDtypeStruct((NCORE, N, DH), jnp.float32)
  else:
    out_spec = pl.BlockSpec((ROWS_MM, DH), lambda i: (i, 0))
    out_shape = jax.ShapeDtypeStruct((N, DH), jnp.float32)
  return pl.pallas_call(
      body, grid=(nblk,), in_specs=in_specs, out_specs=out_spec,
      out_shape=out_shape)(y, s1, s2, g, be)


def _embed_consts():
  S = np.zeros((20, 256), np.float32)
  mA = np.zeros((1, 256), np.float32)
  mB = np.zeros((1, 256), np.float32)
  mC = np.zeros((1, 256), np.float32)
  mD = np.zeros((1, 256), np.float32)
  for g in range(3):
    base = 63 * g
    for j in range(3):
      S[3 * g + j, base + j] = 1.0
      mC[0, base + j] = 1.0
      for i in range(10):
        S[3 * g + j, base + 3 + 6 * i + j] = 2.0**i
        mA[0, base + 3 + 6 * i + j] = 1.0
        S[3 * g + j, base + 6 + 6 * i + j] = 2.0**i
        mB[0, base + 6 + 6 * i + j] = 1.0
  for t in range(11):
    S[9 + t, 189 + t] = 1.0
    mC[0, 189 + t] = 1.0
  mD[0, 200] = 1.0
  return S, mA, mB, mC, mD


_EMB = _embed_consts()


def _embed_split(x):
  def body(x_ref, s_ref, a_ref, b_ref, c_ref, d_ref, o_ref):
    v = x_ref[...]
    pre = jnp.dot(v, s_ref[...], preferred_element_type=jnp.float32,
                  precision=lax.Precision.HIGHEST)
    h = (jnp.sin(pre) * a_ref[...] + jnp.cos(pre) * b_ref[...] +
         pre * c_ref[...] + d_ref[...])
    o_ref[0] = h[:, :DH]
    o_ref[1] = h[:, DH:]

  nblk = N // ROWS_MM
  full = lambda shape: pl.BlockSpec(shape, lambda i: tuple(0 for _ in shape))
  return pl.pallas_call(
      body, grid=(nblk,),
      in_specs=[pl.BlockSpec((ROWS_MM, 20), lambda i: (i, 0)),
                full((20, 256)), full((1, 256)), full((1, 256)),
                full((1, 256)), full((1, 256))],
      out_specs=pl.BlockSpec((2, ROWS_MM, DH), lambda i: (0, i, 0)),
      out_shape=jax.ShapeDtypeStruct((NCORE, N, DH), jnp.float32),
  )(x, *(jnp.asarray(m) for m in _EMB))


def _split_weights(W, d_real, d_pad, feat_split):
  dout = W.shape[0]
  Wa = W[:, :d_real].T
  Wb = W[:, d_real:].T
  A = Wa - Wb
  B = Wb
  if d_pad > d_real:
    A = jnp.pad(A, ((0, d_pad - d_real), (0, 0)))
    B = jnp.pad(B, ((0, d_pad - d_real), (0, 0)))
  if feat_split:
    return A.reshape(2, DH, dout), B.reshape(2, DH, dout)
  return A, B


def kernel(x, edge_index, batch, W1, b1, W2, b2, W3, b3, W4, b4, W5, b5,
           g1, be1, g2, be2, g3, be3, g4, be4):
  del batch
  f32 = jnp.float32
  src = edge_index[0]
  dst = edge_index[1]
  pad = E_PAD - E
  iot = jnp.arange(pad, dtype=jnp.int32)
  srcp = jnp.concatenate([src, iot % 4096])
  dstp = jnp.concatenate([dst, N + iot % N_PAD_ROWS])
  srcp = srcp.reshape(TOT_CHUNK, CHUNK)
  dstp = dstp.reshape(TOT_CHUNK, CHUNK)
  zrows = jnp.zeros((Z_PER_SUB, DH), f32)

  A1, B1 = _split_weights(W1, 200, 256, True)
  A2, B2 = _split_weights(W2, 64, 128, False)
  A3, B3 = _split_weights(W3, 128, 128, False)
  A4, B4 = _split_weights(W4, 256, 256, True)
  A5, B5 = _split_weights(W5, 256, 256, True)
  row = lambda v: v.reshape(1, -1)

  h0 = _embed_split(x)
  nb1 = _segsum_sc(h0, srcp, dstp, zrows, True)
  p1 = _pre_mm(h0, A1, row(b1), 64, True)
  deg = nb1[1, :, 72:73]
  y1, s1, q1 = _post_mm(p1, nb1, deg, B1, 64, True, True)
  h1 = _bn_relu(y1, s1, q1, row(g1), row(be1), 64, False)
  nb2 = _segsum_sc(h1, srcp, dstp, zrows, False)
  p2 = _pre_mm(h1, A2, row(b2), 128, False)
  y2, s2, q2 = _post_mm(p2, nb2, deg, B2, 128, True, False)
  h2 = _bn_relu(y2, s2, q2, row(g2), row(be2), 128, False)
  nb3 = _segsum_sc(h2, srcp, dstp, zrows, False)
  p3 = _pre_mm(h2, A3, row(b3), 256, False)
  y3, s3, q3 = _post_mm(p3, nb3, deg, B3, 256, True, False)
  h3 = _bn_relu(y3, s3, q3, row(g3), row(be3), 256, True)
  nb4 = _segsum_sc(h3, srcp, dstp, zrows, True)
  p4 = _pre_mm(h3, A4, row(b4), 256, True)
  y4, s4, q4 = _post_mm(p4, nb4, deg, B4, 256, True, True)
  h4 = _bn_relu(y4, s4, q4, row(g4), row(be4), 256, True)
  nb5 = _segsum_sc(h4, srcp, dstp, zrows, True)
  p5 = _pre_mm(h4, A5, row(b5), 512, True)
  (y5,) = _post_mm(p5, nb5, deg, B5, 512, False, True)
  return y5
